# baseline (speedup 1.0000x reference)
import functools

import numpy as np
import jax
import jax.numpy as jnp
from jax import lax
from jax.experimental import pallas as pl
from jax.experimental.pallas import tpu as pltpu
from jax.experimental.pallas import tpu_sc as plsc

N = 10000
E = 320000
F = 128
H = F // 2

NC = 2
NS = 16
L = 16
EPW = E // NS
C = 40
NCHUNK = EPW // C
NB = 5
KPF = 2
RPT = 632
PADN = NS * RPT

_LOG2 = 0.6931471805599453


def _mlp_body(edges_ref, w1_ref, b1_ref, w2_ref, b2_ref, out_ref):
    x = jnp.dot(edges_ref[...].astype(jnp.bfloat16), w1_ref[...],
                preferred_element_type=jnp.float32)
    x = x + b1_ref[...]
    x = jnp.log(1.0 + jnp.exp(-jnp.abs(x))) + jnp.maximum(x, 0.0) - _LOG2
    xb = x.astype(jnp.bfloat16)
    out_ref[...] = jnp.dot(xb, w2_ref[...],
                           preferred_element_type=jnp.float32) + b2_ref[...]


def _mlp(edges, W1, b1, W2, b2):
    BLK = 3200
    pc = pl.pallas_call(
        _mlp_body,
        grid=(E // BLK,),
        in_specs=[
            pl.BlockSpec((BLK, F), lambda i: (i, 0)),
            pl.BlockSpec((F, F), lambda i: (0, 0)),
            pl.BlockSpec((1, F), lambda i: (0, 0)),
            pl.BlockSpec((F, F), lambda i: (0, 0)),
            pl.BlockSpec((1, F), lambda i: (0, 0)),
        ],
        out_specs=pl.BlockSpec((BLK, F), lambda i: (i, 0)),
        out_shape=jax.ShapeDtypeStruct((E, F), jnp.float32),
    )
    return pc(edges, W1.astype(jnp.bfloat16), b1.reshape(1, F),
              W2.astype(jnp.bfloat16), b2.reshape(1, F))


def _sc_body(filt_hbm, nodes_hbm, src_hbm, dst_hbm, zeros_hbm, out_hbm,
             src_all, dst_all, rows, filtb, acc_sh, *sems):
    sem_g = sems[0:NB]
    sem_f = sems[NB:2 * NB]
    sem_s = sems[2 * NB:3 * NB]
    cid = lax.axis_index("c")
    sid = lax.axis_index("s")

    pltpu.async_copy(src_hbm.at[cid, sid], src_all, sem_g[0])
    pltpu.async_copy(dst_hbm.at[sid], dst_all, sem_f[0])

    row0 = sid * RPT
    pltpu.sync_copy(zeros_hbm, acc_sh.at[pl.ds(row0, RPT)])

    pltpu.make_async_copy(src_hbm.at[cid, sid], src_all, sem_g[0]).wait()
    pltpu.make_async_copy(dst_hbm.at[sid], dst_all, sem_f[0]).wait()

    def issue_fetch(u, b):
        pltpu.async_copy(nodes_hbm.at[src_all.at[u]], rows.at[b], sem_g[b])
        pltpu.async_copy(filt_hbm.at[pl.ds(sid * EPW + u * C, C)],
                         filtb.at[b], sem_f[b])

    for b in range(KPF):
        issue_fetch(b, b)
    plsc.subcore_barrier()

    def step(t, b):
        bpf = (b + KPF) % NB

        @pl.when(t < NCHUNK - KPF)
        def _():
            @pl.when(t >= NB - KPF)
            def _():
                pltpu.make_async_copy(
                    rows.at[bpf], acc_sh.at[dst_all.at[t + KPF - NB]],
                    sem_s[bpf]).wait()
            issue_fetch(t + KPF, bpf)

        pltpu.make_async_copy(nodes_hbm.at[src_all.at[t]], rows.at[b],
                              sem_g[b]).wait()
        pltpu.make_async_copy(filt_hbm.at[pl.ds(sid * EPW + t * C, C)],
                              filtb.at[b], sem_f[b]).wait()

        rows_b = rows.at[b]
        filt_b = filtb.at[b]
        col0 = cid * H

        def mul(i, _):
            for j in range(H // L):
                sl = pl.ds(j * L, L)
                rows_b[i, sl] = (rows_b[i, sl]
                                 * filt_b[i, pl.ds(col0 + j * L, L)])
            return 0

        lax.fori_loop(0, C, mul, 0)

        @pl.when(t < NCHUNK - NB)
        def _():
            pltpu.async_copy(rows_b, acc_sh.at[dst_all.at[t]], sem_s[b],
                             add=True)

        @pl.when(t >= NCHUNK - NB)
        def _():
            pltpu.sync_copy(rows_b, acc_sh.at[dst_all.at[t]], add=True)

    def group(g, _):
        for b in range(NB):
            step(g * NB + b, b)
        return 0

    lax.fori_loop(0, NCHUNK // NB, group, 0)
    plsc.subcore_barrier()

    pltpu.sync_copy(acc_sh.at[pl.ds(row0, RPT)],
                    out_hbm.at[cid, pl.ds(row0, RPT)])


def _sc_scatter(filt2, nodes2, src2, dst):
    mesh = plsc.VectorSubcoreMesh(core_axis_name="c", subcore_axis_name="s",
                                  num_cores=NC, num_subcores=NS)
    f = functools.partial(
        pl.kernel,
        out_type=jax.ShapeDtypeStruct((NC, PADN, H), jnp.float32),
        mesh=mesh,
        compiler_params=pltpu.CompilerParams(use_tc_tiling_on_sc=False),
        scratch_types=[
            pltpu.VMEM((NCHUNK, C), jnp.int32),
            pltpu.VMEM((NCHUNK, C), jnp.int32),
            pltpu.VMEM((NB, C, H), jnp.float32),
            pltpu.VMEM((NB, C, F), jnp.float32),
            pltpu.VMEM_SHARED((PADN, H), jnp.float32),
        ] + [pltpu.SemaphoreType.DMA] * (3 * NB),
    )(_sc_body)
    zeros = jnp.zeros((RPT, H), jnp.float32)
    return f(filt2, nodes2, src2, dst.reshape(NS, NCHUNK, C), zeros)


def kernel(nodes, edges, edge_index, W1, b1, W2, b2):
    filt2 = _mlp(edges, W1, b1, W2, b2)
    dst = edge_index[0]
    src = edge_index[1].reshape(NS, NCHUNK, C)
    nodes2 = nodes.reshape(NC * N, H)
    src2 = jnp.stack([2 * src, 2 * src + 1])
    partials = _sc_scatter(filt2, nodes2, src2, dst)
    return jnp.concatenate([partials[0, :N], partials[1, :N]], axis=-1)


# speedup vs baseline: 1.4156x; 1.4156x over previous
import functools

import numpy as np
import jax
import jax.numpy as jnp
from jax import lax
from jax.experimental import pallas as pl
from jax.experimental.pallas import tpu as pltpu
from jax.experimental.pallas import tpu_sc as plsc

N = 10000
E = 320000
F = 128
H = F // 2

NC = 2
NS = 16
L = 16
EPW = E // NS
C = 40
NCHUNK = EPW // C
NB = 5
KPF = 2
RPT = 632
PADN = NS * RPT

_LOG2 = 0.6931471805599453


def _mlp_body(edges_ref, w1_ref, b1_ref, w2_ref, b2_ref, out_ref):
    x = jnp.dot(edges_ref[...].astype(jnp.bfloat16), w1_ref[...],
                preferred_element_type=jnp.float32)
    x = x + b1_ref[...]
    x = jnp.log(1.0 + jnp.exp(-jnp.abs(x))) + jnp.maximum(x, 0.0) - _LOG2
    xb = x.astype(jnp.bfloat16)
    y = jnp.dot(xb, w2_ref[...],
                preferred_element_type=jnp.float32) + b2_ref[...]
    y2 = y.reshape(y.shape[0] // 2, 2, F)
    out_ref[0] = jnp.concatenate([y2[:, 0, :H], y2[:, 1, :H]], axis=-1)
    out_ref[1] = jnp.concatenate([y2[:, 0, H:], y2[:, 1, H:]], axis=-1)


def _mlp(edges, W1, b1, W2, b2):
    BLK = 3200
    pc = pl.pallas_call(
        _mlp_body,
        grid=(E // BLK,),
        in_specs=[
            pl.BlockSpec((BLK, F), lambda i: (i, 0)),
            pl.BlockSpec((F, F), lambda i: (0, 0)),
            pl.BlockSpec((1, F), lambda i: (0, 0)),
            pl.BlockSpec((F, F), lambda i: (0, 0)),
            pl.BlockSpec((1, F), lambda i: (0, 0)),
        ],
        out_specs=pl.BlockSpec((NC, BLK // 2, F), lambda i: (0, i, 0)),
        out_shape=jax.ShapeDtypeStruct((NC, E // 2, F), jnp.float32),
    )
    return pc(edges, W1.astype(jnp.bfloat16), b1.reshape(1, F),
              W2.astype(jnp.bfloat16), b2.reshape(1, F))


def _sc_body(filt_hbm, nodes_hbm, src_hbm, dst_hbm, zeros_hbm, out_hbm,
             src_all, dst_all, rows, filtb, acc_sh, *sems):
    sem_g = sems[0:NB]
    sem_f = sems[NB:2 * NB]
    sem_s = sems[2 * NB:3 * NB]
    cid = lax.axis_index("c")
    sid = lax.axis_index("s")

    pltpu.async_copy(src_hbm.at[cid, sid], src_all, sem_g[0])
    pltpu.async_copy(dst_hbm.at[sid], dst_all, sem_f[0])

    row0 = sid * RPT
    pltpu.sync_copy(zeros_hbm, acc_sh.at[pl.ds(row0, RPT)])

    pltpu.make_async_copy(src_hbm.at[cid, sid], src_all, sem_g[0]).wait()
    pltpu.make_async_copy(dst_hbm.at[sid], dst_all, sem_f[0]).wait()

    def issue_fetch(u, b):
        pltpu.async_copy(nodes_hbm.at[src_all.at[u]], rows.at[b], sem_g[b])
        pltpu.async_copy(filt_hbm.at[cid, pl.ds(sid * (EPW // 2) + u * (C // 2),
                                                 C // 2)],
                         filtb.at[b], sem_f[b])

    for b in range(KPF):
        issue_fetch(b, b)
    plsc.subcore_barrier()

    def step(t, b):
        bpf = (b + KPF) % NB

        @pl.when(t < NCHUNK - KPF)
        def _():
            @pl.when(t >= NB - KPF)
            def _():
                pltpu.make_async_copy(
                    rows.at[bpf], acc_sh.at[dst_all.at[t + KPF - NB]],
                    sem_s[bpf]).wait()
            issue_fetch(t + KPF, bpf)

        pltpu.make_async_copy(nodes_hbm.at[src_all.at[t]], rows.at[b],
                              sem_g[b]).wait()
        pltpu.make_async_copy(filt_hbm.at[cid, pl.ds(sid * (EPW // 2)
                                                      + t * (C // 2), C // 2)],
                              filtb.at[b], sem_f[b]).wait()

        rows_b = rows.at[b]
        filt_b = filtb.at[b]

        def mul(p, _):
            i0 = 2 * p
            for j in range(H // L):
                sl = pl.ds(j * L, L)
                rows_b[i0, sl] = rows_b[i0, sl] * filt_b[p, sl]
                rows_b[i0 + 1, sl] = (rows_b[i0 + 1, sl]
                                      * filt_b[p, pl.ds(H + j * L, L)])
            return 0

        lax.fori_loop(0, C // 2, mul, 0)

        @pl.when(t < NCHUNK - NB)
        def _():
            pltpu.async_copy(rows_b, acc_sh.at[dst_all.at[t]], sem_s[b],
                             add=True)

        @pl.when(t >= NCHUNK - NB)
        def _():
            pltpu.sync_copy(rows_b, acc_sh.at[dst_all.at[t]], add=True)

    def group(g, _):
        for b in range(NB):
            step(g * NB + b, b)
        return 0

    lax.fori_loop(0, NCHUNK // NB, group, 0)
    plsc.subcore_barrier()

    pltpu.sync_copy(acc_sh.at[pl.ds(row0, RPT)],
                    out_hbm.at[cid, pl.ds(row0, RPT)])


def _sc_scatter(filt2, nodes2, src2, dst):
    mesh = plsc.VectorSubcoreMesh(core_axis_name="c", subcore_axis_name="s",
                                  num_cores=NC, num_subcores=NS)
    f = functools.partial(
        pl.kernel,
        out_type=jax.ShapeDtypeStruct((NC, PADN, H), jnp.float32),
        mesh=mesh,
        compiler_params=pltpu.CompilerParams(use_tc_tiling_on_sc=False),
        scratch_types=[
            pltpu.VMEM((NCHUNK, C), jnp.int32),
            pltpu.VMEM((NCHUNK, C), jnp.int32),
            pltpu.VMEM((NB, C, H), jnp.float32),
            pltpu.VMEM((NB, C // 2, F), jnp.float32),
            pltpu.VMEM_SHARED((PADN, H), jnp.float32),
        ] + [pltpu.SemaphoreType.DMA] * (3 * NB),
    )(_sc_body)
    zeros = jnp.zeros((RPT, H), jnp.float32)
    return f(filt2, nodes2, src2, dst.reshape(NS, NCHUNK, C), zeros)


def kernel(nodes, edges, edge_index, W1, b1, W2, b2):
    filt2 = _mlp(edges, W1, b1, W2, b2)
    dst = edge_index[0]
    src = edge_index[1].reshape(NS, NCHUNK, C)
    nodes2 = nodes.reshape(NC * N, H)
    src2 = jnp.stack([2 * src, 2 * src + 1])
    partials = _sc_scatter(filt2, nodes2, src2, dst)
    return jnp.concatenate([partials[0, :N], partials[1, :N]], axis=-1)
